# baseline (speedup 1.0000x reference)
import functools

import jax
import jax.numpy as jnp
from jax import lax
from jax.experimental import pallas as pl
from jax.experimental.pallas import tpu as pltpu
from jax.experimental.pallas import tpu_sc as plsc

N = 10000
D = 128
H = 8
C = 8
NCLS = 64
E = 320000

NPAD = 10240
ROWS_PER_TILE = NPAD // 32
ETOT = E + N
EPAD = 344064
K1 = 256
KQ1 = K1 // 128
CHUNKS = EPAD // K1
CPT1 = CHUNKS // 32

_f32 = jnp.float32
_i32 = jnp.int32


def _dense1_body(x_ref, w_ref, a_ref, xl_ref, asd_ref):
    xl = jnp.dot(x_ref[...], w_ref[...], preferred_element_type=_f32)
    xl_ref[...] = xl
    asd_ref[...] = jnp.dot(xl, a_ref[...], preferred_element_type=_f32)


def _mid_body(p64_ref, p8_ref, b1_ref, w2_ref, a2_ref, e8_ref, xl2_ref, asd2_ref):
    p64 = p64_ref[0] + p64_ref[1]
    den = p8_ref[0][:, 0:8] + p8_ref[1][:, 0:8]
    dexp = jnp.dot(den, e8_ref[...], preferred_element_type=_f32)
    h = p64 / (dexp + 1e-16) + b1_ref[...]
    h = jnp.where(h > 0, h, jnp.exp(jnp.minimum(h, 0.0)) - 1.0)
    xl2 = jnp.dot(h, w2_ref[...], preferred_element_type=_f32)
    xl2_ref[...] = xl2
    asd2_ref[...] = jnp.dot(xl2, a2_ref[...], preferred_element_type=_f32)


def _fin_body(p64_ref, p8_ref, b2_ref, out_ref):
    p64 = p64_ref[0] + p64_ref[1]
    den = p8_ref[0][:, 0:1] + p8_ref[1][:, 0:1]
    out_ref[...] = p64 / (den + 1e-16) + b2_ref[...]


_MESH = plsc.VectorSubcoreMesh(
    core_axis_name="c", subcore_axis_name="s", num_cores=2, num_subcores=16
)


def _zero_rows(ref, nrows, width):
    z16 = jnp.zeros((16,), _f32)

    def body(r, _):
        for o in range(0, width, 16):
            ref[r, pl.ds(o, 16)] = z16
        return 0

    lax.fori_loop(0, nrows, body, 0)


def _sc_edge_pass(xl_hbm, asd_hbm, e3_hbm, out64_hbm, out8_hbm,
                  sbufA, dbufA, xrA, wbufA, e3A,
                  sbufB, dbufB, xrB, wbufB, e3B,
                  acc64, acc8, gsemA, gsemB, ssemA, ssemB):
    cid = lax.axis_index("c")
    sid = lax.axis_index("s")
    wid = cid * 16 + sid

    lane = lax.iota(_i32, 16)
    c1 = (lane >= 8).astype(_i32)
    hvec = lane - 8 * c1
    hw = ROWS_PER_TILE // 2

    _zero_rows(xrA, K1, 64)
    _zero_rows(wbufA, K1, 16)
    for t in range(2):
        half = pl.ds(sid * ROWS_PER_TILE + t * hw, hw)
        pltpu.sync_copy(xrA.at[pl.ds(0, hw)], acc64.at[half])
        pltpu.sync_copy(wbufA.at[pl.ds(0, hw)], acc8.at[half])
    plsc.subcore_barrier()

    def gather_descs(e3, sbuf, dbuf, xr, gsem):
        descs = []
        for q in range(KQ1):
            sl = pl.ds(128 * q, 128)
            descs.append(pltpu.make_async_copy(asd_hbm.at[e3.at[q]], sbuf.at[sl], gsem))
            descs.append(pltpu.make_async_copy(asd_hbm.at[e3.at[KQ1 + q]], dbuf.at[sl], gsem))
            descs.append(pltpu.make_async_copy(xl_hbm.at[e3.at[q]], xr.at[sl], gsem))
        return descs

    def scatter_descs(e3, xr, wbuf, ssem):
        descs = []
        for q in range(KQ1):
            sl = pl.ds(128 * q, 128)
            descs.append(pltpu.make_async_copy(xr.at[sl], acc64.at[e3.at[KQ1 + q]], ssem))
            descs.append(pltpu.make_async_copy(wbuf.at[sl], acc8.at[e3.at[KQ1 + q]], ssem))
        return descs

    def fire_gathers(c, e3, sbuf, dbuf, xr, gsem):
        pltpu.sync_copy(e3_hbm.at[wid + 32 * c], e3)
        for dsc in gather_descs(e3, sbuf, dbuf, xr, gsem):
            dsc.start()

    def wait_gathers(e3, sbuf, dbuf, xr, gsem):
        for dsc in gather_descs(e3, sbuf, dbuf, xr, gsem):
            dsc.wait()

    def fire_scatters(e3, xr, wbuf, ssem):
        for dsc in scatter_descs(e3, xr, wbuf, ssem):
            dsc.start(add=True)

    def wait_scatters(e3, xr, wbuf, ssem):
        for dsc in scatter_descs(e3, xr, wbuf, ssem):
            dsc.wait()

    _dnums = lax.GatherDimensionNumbers(
        offset_dims=(), collapsed_slice_dims=(0,), start_index_map=(0,)
    )

    def _shuffle(x, idx):
        return lax.gather(x, idx[:, None], _dnums, slice_sizes=(1,),
                          mode=lax.GatherScatterMode.PROMISE_IN_BOUNDS)

    def compute(e3, sbuf, dbuf, xr, wbuf):
        @plsc.parallel_loop(0, K1 // 2, unroll=4)
        def _body(i):
            e = 2 * i + c1
            av = plsc.load_gather(sbuf, [e, hvec])
            dv = plsc.load_gather(dbuf, [e, 8 + hvec])
            m = plsc.load_gather(e3, [2 * KQ1 + (e >> 7), e & 127])
            a = av + dv
            a = jnp.maximum(a, 0.2 * a)
            w = plsc.bitcast(m, _f32) * jnp.exp(a)
            plsc.store_scatter(wbuf, [e, hvec], w)
            for u in range(2):
                eu = 2 * i + u
                for g in range(4):
                    wv = _shuffle(w, 8 * u + 2 * g + c1)
                    xr[eu, pl.ds(16 * g, 16)] = xr[eu, pl.ds(16 * g, 16)] * wv

    fire_gathers(0, e3A, sbufA, dbufA, xrA, gsemA)
    fire_gathers(1, e3B, sbufB, dbufB, xrB, gsemB)

    def pair_body(ii, _):
        wait_gathers(e3A, sbufA, dbufA, xrA, gsemA)
        compute(e3A, sbufA, dbufA, xrA, wbufA)
        fire_scatters(e3A, xrA, wbufA, ssemA)

        wait_gathers(e3B, sbufB, dbufB, xrB, gsemB)
        compute(e3B, sbufB, dbufB, xrB, wbufB)
        fire_scatters(e3B, xrB, wbufB, ssemB)

        wait_scatters(e3A, xrA, wbufA, ssemA)

        @pl.when(ii + 1 < CPT1 // 2)
        def _():
            fire_gathers(2 * ii + 2, e3A, sbufA, dbufA, xrA, gsemA)

        wait_scatters(e3B, xrB, wbufB, ssemB)

        @pl.when(ii + 1 < CPT1 // 2)
        def _():
            fire_gathers(2 * ii + 3, e3B, sbufB, dbufB, xrB, gsemB)

        return 0

    lax.fori_loop(0, CPT1 // 2, pair_body, 0)

    plsc.subcore_barrier()
    for t in range(2):
        r0 = sid * ROWS_PER_TILE + t * hw
        pltpu.sync_copy(acc64.at[pl.ds(r0, hw)], xrA.at[pl.ds(0, hw)])
        pltpu.sync_copy(xrA.at[pl.ds(0, hw)], out64_hbm.at[cid, pl.ds(r0, hw)])
        pltpu.sync_copy(acc8.at[pl.ds(r0, hw)], wbufA.at[pl.ds(0, hw)])
        pltpu.sync_copy(wbufA.at[pl.ds(0, hw)], out8_hbm.at[cid, pl.ds(r0, hw)])


_sc1 = functools.partial(
    pl.kernel,
    out_type=[
        jax.ShapeDtypeStruct((2, NPAD, 64), _f32),
        jax.ShapeDtypeStruct((2, NPAD, 16), _f32),
    ],
    mesh=_MESH,
    scratch_types=[
        pltpu.VMEM((K1, 16), _f32),
        pltpu.VMEM((K1, 16), _f32),
        pltpu.VMEM((K1, 64), _f32),
        pltpu.VMEM((K1, 16), _f32),
        pltpu.VMEM((3 * KQ1, 128), _i32),
        pltpu.VMEM((K1, 16), _f32),
        pltpu.VMEM((K1, 16), _f32),
        pltpu.VMEM((K1, 64), _f32),
        pltpu.VMEM((K1, 16), _f32),
        pltpu.VMEM((3 * KQ1, 128), _i32),
        pltpu.VMEM_SHARED((NPAD, 64), _f32),
        pltpu.VMEM_SHARED((NPAD, 16), _f32),
        pltpu.SemaphoreType.DMA,
        pltpu.SemaphoreType.DMA,
        pltpu.SemaphoreType.DMA,
        pltpu.SemaphoreType.DMA,
    ],
    compiler_params=pltpu.CompilerParams(
        use_tc_tiling_on_sc=False, needs_layout_passes=False
    ),
)(_sc_edge_pass)


@jax.jit
def kernel(x, edge_index, W1, att_src1, att_dst1, b1, W2, att_src2, att_dst2, b2):
    src0 = edge_index[0]
    dst0 = edge_index[1]
    loop = jnp.arange(N, dtype=_i32)
    padn = EPAD - ETOT
    pad_i = jnp.arange(padn, dtype=_i32) % N
    src = jnp.concatenate([src0, loop, pad_i])
    dst = jnp.concatenate([dst0, loop, pad_i])
    maskf = jnp.concatenate([
        (src0 != dst0).astype(_f32),
        jnp.ones((N,), _f32),
        jnp.zeros((padn,), _f32),
    ])
    mbits = lax.bitcast_convert_type(maskf, _i32)
    e3 = jnp.concatenate([
        src.reshape(CHUNKS, KQ1, 128),
        dst.reshape(CHUNKS, KQ1, 128),
        mbits.reshape(CHUNKS, KQ1, 128),
    ], axis=1)

    eyeH = jnp.eye(H, dtype=_f32)
    Ms = (att_src1[0][:, :, None] * eyeH[:, None, :]).reshape(H * C, H)
    Md = (att_dst1[0][:, :, None] * eyeH[:, None, :]).reshape(H * C, H)
    A1 = jnp.concatenate([Ms, Md], axis=1)
    A2 = jnp.concatenate([
        jnp.tile(att_src2[0, 0][:, None], (1, H)),
        jnp.tile(att_dst2[0, 0][:, None], (1, H)),
    ], axis=1)
    E8 = jnp.repeat(eyeH, C, axis=1)
    b1r = b1.reshape(1, H * C)
    b2r = b2.reshape(1, NCLS)

    BN1 = 2000
    xl1, asd1 = pl.pallas_call(
        _dense1_body,
        grid=(N // BN1,),
        in_specs=[
            pl.BlockSpec((BN1, D), lambda i: (i, 0)),
            pl.BlockSpec((D, H * C), lambda i: (0, 0)),
            pl.BlockSpec((H * C, 2 * H), lambda i: (0, 0)),
        ],
        out_specs=[
            pl.BlockSpec((BN1, H * C), lambda i: (i, 0)),
            pl.BlockSpec((BN1, 2 * H), lambda i: (i, 0)),
        ],
        out_shape=[
            jax.ShapeDtypeStruct((N, H * C), _f32),
            jax.ShapeDtypeStruct((N, 2 * H), _f32),
        ],
    )(x, W1, A1)

    p64_1, p8_1 = _sc1(xl1, asd1, e3)

    BN2 = 2560
    xl2, asd2 = pl.pallas_call(
        _mid_body,
        grid=(NPAD // BN2,),
        in_specs=[
            pl.BlockSpec((2, BN2, 64), lambda i: (0, i, 0)),
            pl.BlockSpec((2, BN2, 16), lambda i: (0, i, 0)),
            pl.BlockSpec((1, H * C), lambda i: (0, 0)),
            pl.BlockSpec((H * C, NCLS), lambda i: (0, 0)),
            pl.BlockSpec((NCLS, 2 * H), lambda i: (0, 0)),
            pl.BlockSpec((H, H * C), lambda i: (0, 0)),
        ],
        out_specs=[
            pl.BlockSpec((BN2, NCLS), lambda i: (i, 0)),
            pl.BlockSpec((BN2, 2 * H), lambda i: (i, 0)),
        ],
        out_shape=[
            jax.ShapeDtypeStruct((NPAD, NCLS), _f32),
            jax.ShapeDtypeStruct((NPAD, 2 * H), _f32),
        ],
    )(p64_1, p8_1, b1r, W2, A2, E8)

    p64_2, p8_2 = _sc1(xl2[:N], asd2[:N], e3)

    out = pl.pallas_call(
        _fin_body,
        grid=(NPAD // BN2,),
        in_specs=[
            pl.BlockSpec((2, BN2, 64), lambda i: (0, i, 0)),
            pl.BlockSpec((2, BN2, 16), lambda i: (0, i, 0)),
            pl.BlockSpec((1, NCLS), lambda i: (0, 0)),
        ],
        out_specs=pl.BlockSpec((BN2, NCLS), lambda i: (i, 0)),
        out_shape=jax.ShapeDtypeStruct((NPAD, NCLS), _f32),
    )(p64_2, p8_2, b2r)

    return out[:N]


# speedup vs baseline: 1.0347x; 1.0347x over previous
import functools

import jax
import jax.numpy as jnp
from jax import lax
from jax.experimental import pallas as pl
from jax.experimental.pallas import tpu as pltpu
from jax.experimental.pallas import tpu_sc as plsc

N = 10000
D = 128
H = 8
C = 8
NCLS = 64
E = 320000

NPAD = 10240
ROWS_PER_TILE = NPAD // 32
ETOT = E + N
EPAD = 344064
K1 = 256
KQ1 = K1 // 128
CHUNKS = EPAD // K1
CPT1 = CHUNKS // 32

_f32 = jnp.float32
_i32 = jnp.int32


def _dense1_body(x_ref, w_ref, a_ref, xl_ref, asd_ref):
    xl = jnp.dot(x_ref[...], w_ref[...], preferred_element_type=_f32)
    xl_ref[...] = xl
    asd_ref[...] = jnp.dot(xl, a_ref[...], preferred_element_type=_f32)


def _mid_body(p64_ref, p8_ref, b1_ref, w2_ref, a2_ref, e8_ref, xl2_ref, asd2_ref):
    p64 = p64_ref[0] + p64_ref[1]
    den = p8_ref[0][:, 0:8] + p8_ref[1][:, 0:8]
    dexp = jnp.dot(den, e8_ref[...], preferred_element_type=_f32)
    h = p64 / (dexp + 1e-16) + b1_ref[...]
    h = jnp.where(h > 0, h, jnp.exp(jnp.minimum(h, 0.0)) - 1.0)
    xl2 = jnp.dot(h, w2_ref[...], preferred_element_type=_f32)
    xl2_ref[...] = xl2
    asd2_ref[...] = jnp.dot(xl2, a2_ref[...], preferred_element_type=_f32)


def _fin_body(p64_ref, p8_ref, b2_ref, out_ref):
    p64 = p64_ref[0] + p64_ref[1]
    den = p8_ref[0][:, 0:1] + p8_ref[1][:, 0:1]
    out_ref[...] = p64 / (den + 1e-16) + b2_ref[...]


_MESH = plsc.VectorSubcoreMesh(
    core_axis_name="c", subcore_axis_name="s", num_cores=2, num_subcores=16
)


def _zero_rows(ref, nrows, width):
    z16 = jnp.zeros((16,), _f32)

    def body(r, _):
        for o in range(0, width, 16):
            ref[r, pl.ds(o, 16)] = z16
        return 0

    lax.fori_loop(0, nrows, body, 0)


def _sc_edge_pass(xl_hbm, asd_hbm, e3_hbm, out64_hbm, out8_hbm,
                  sbufA, dbufA, xrA, wbufA, e3A,
                  sbufB, dbufB, xrB, wbufB, e3B,
                  acc64, acc8, gsemA, gsemB, ssemA, ssemB):
    cid = lax.axis_index("c")
    sid = lax.axis_index("s")
    wid = cid * 16 + sid

    lane = lax.iota(_i32, 16)
    c1 = (lane >= 8).astype(_i32)
    hvec = lane - 8 * c1
    hw = ROWS_PER_TILE // 2

    _zero_rows(xrA, K1, 64)
    _zero_rows(wbufA, K1, 16)
    for t in range(2):
        half = pl.ds(sid * ROWS_PER_TILE + t * hw, hw)
        pltpu.sync_copy(xrA.at[pl.ds(0, hw)], acc64.at[half])
        pltpu.sync_copy(wbufA.at[pl.ds(0, hw)], acc8.at[half])
    plsc.subcore_barrier()

    def gather_descs(e3, sbuf, dbuf, xr, gsem):
        descs = []
        for q in range(KQ1):
            sl = pl.ds(128 * q, 128)
            descs.append(pltpu.make_async_copy(asd_hbm.at[e3.at[q]], sbuf.at[sl], gsem))
            descs.append(pltpu.make_async_copy(asd_hbm.at[e3.at[KQ1 + q]], dbuf.at[sl], gsem))
            descs.append(pltpu.make_async_copy(xl_hbm.at[e3.at[q]], xr.at[sl], gsem))
        return descs

    def scatter_descs(e3, xr, wbuf, ssem):
        descs = []
        for q in range(KQ1):
            sl = pl.ds(128 * q, 128)
            descs.append(pltpu.make_async_copy(xr.at[sl], acc64.at[e3.at[KQ1 + q]], ssem))
            descs.append(pltpu.make_async_copy(wbuf.at[sl], acc8.at[e3.at[KQ1 + q]], ssem))
        return descs

    def fire_gathers(c, e3, sbuf, dbuf, xr, gsem):
        pltpu.sync_copy(e3_hbm.at[wid + 32 * c], e3)
        for dsc in gather_descs(e3, sbuf, dbuf, xr, gsem):
            dsc.start()

    def wait_gathers(e3, sbuf, dbuf, xr, gsem):
        for dsc in gather_descs(e3, sbuf, dbuf, xr, gsem):
            dsc.wait()

    def fire_scatters(e3, xr, wbuf, ssem):
        for dsc in scatter_descs(e3, xr, wbuf, ssem):
            dsc.start(add=True)

    def wait_scatters(e3, xr, wbuf, ssem):
        for dsc in scatter_descs(e3, xr, wbuf, ssem):
            dsc.wait()

    _dnums = lax.GatherDimensionNumbers(
        offset_dims=(), collapsed_slice_dims=(0,), start_index_map=(0,)
    )

    def _shuffle(x, idx):
        return lax.gather(x, idx[:, None], _dnums, slice_sizes=(1,),
                          mode=lax.GatherScatterMode.PROMISE_IN_BOUNDS)

    def compute(e3, sbuf, dbuf, xr, wbuf):
        @plsc.parallel_loop(0, K1 // 2, unroll=2)
        def _body(i):
            e = 2 * i + c1
            av = plsc.load_gather(sbuf, [e, hvec])
            dv = plsc.load_gather(dbuf, [e, 8 + hvec])
            m = plsc.load_gather(e3, [2 * KQ1 + (e >> 7), e & 127])
            a = av + dv
            a = jnp.maximum(a, 0.2 * a)
            w = plsc.bitcast(m, _f32) * jnp.exp(a)
            plsc.store_scatter(wbuf, [e, hvec], w)
            for u in range(2):
                eu = 2 * i + u
                for g in range(4):
                    wv = _shuffle(w, 8 * u + 2 * g + c1)
                    xr[eu, pl.ds(16 * g, 16)] = xr[eu, pl.ds(16 * g, 16)] * wv

    fire_gathers(0, e3A, sbufA, dbufA, xrA, gsemA)
    fire_gathers(1, e3B, sbufB, dbufB, xrB, gsemB)

    def pair_body(ii, _):
        wait_gathers(e3A, sbufA, dbufA, xrA, gsemA)
        compute(e3A, sbufA, dbufA, xrA, wbufA)
        fire_scatters(e3A, xrA, wbufA, ssemA)

        wait_gathers(e3B, sbufB, dbufB, xrB, gsemB)
        compute(e3B, sbufB, dbufB, xrB, wbufB)
        fire_scatters(e3B, xrB, wbufB, ssemB)

        wait_scatters(e3A, xrA, wbufA, ssemA)

        @pl.when(ii + 1 < CPT1 // 2)
        def _():
            fire_gathers(2 * ii + 2, e3A, sbufA, dbufA, xrA, gsemA)

        wait_scatters(e3B, xrB, wbufB, ssemB)

        @pl.when(ii + 1 < CPT1 // 2)
        def _():
            fire_gathers(2 * ii + 3, e3B, sbufB, dbufB, xrB, gsemB)

        return 0

    lax.fori_loop(0, CPT1 // 2, pair_body, 0)

    plsc.subcore_barrier()
    for t in range(2):
        r0 = sid * ROWS_PER_TILE + t * hw
        pltpu.sync_copy(acc64.at[pl.ds(r0, hw)], xrA.at[pl.ds(0, hw)])
        pltpu.sync_copy(xrA.at[pl.ds(0, hw)], out64_hbm.at[cid, pl.ds(r0, hw)])
        pltpu.sync_copy(acc8.at[pl.ds(r0, hw)], wbufA.at[pl.ds(0, hw)])
        pltpu.sync_copy(wbufA.at[pl.ds(0, hw)], out8_hbm.at[cid, pl.ds(r0, hw)])


_sc1 = functools.partial(
    pl.kernel,
    out_type=[
        jax.ShapeDtypeStruct((2, NPAD, 64), _f32),
        jax.ShapeDtypeStruct((2, NPAD, 16), _f32),
    ],
    mesh=_MESH,
    scratch_types=[
        pltpu.VMEM((K1, 16), _f32),
        pltpu.VMEM((K1, 16), _f32),
        pltpu.VMEM((K1, 64), _f32),
        pltpu.VMEM((K1, 16), _f32),
        pltpu.VMEM((3 * KQ1, 128), _i32),
        pltpu.VMEM((K1, 16), _f32),
        pltpu.VMEM((K1, 16), _f32),
        pltpu.VMEM((K1, 64), _f32),
        pltpu.VMEM((K1, 16), _f32),
        pltpu.VMEM((3 * KQ1, 128), _i32),
        pltpu.VMEM_SHARED((NPAD, 64), _f32),
        pltpu.VMEM_SHARED((NPAD, 16), _f32),
        pltpu.SemaphoreType.DMA,
        pltpu.SemaphoreType.DMA,
        pltpu.SemaphoreType.DMA,
        pltpu.SemaphoreType.DMA,
    ],
    compiler_params=pltpu.CompilerParams(
        use_tc_tiling_on_sc=False, needs_layout_passes=False
    ),
)(_sc_edge_pass)


@jax.jit
def kernel(x, edge_index, W1, att_src1, att_dst1, b1, W2, att_src2, att_dst2, b2):
    src0 = edge_index[0]
    dst0 = edge_index[1]
    loop = jnp.arange(N, dtype=_i32)
    padn = EPAD - ETOT
    pad_i = jnp.arange(padn, dtype=_i32) % N
    src = jnp.concatenate([src0, loop, pad_i])
    dst = jnp.concatenate([dst0, loop, pad_i])
    maskf = jnp.concatenate([
        (src0 != dst0).astype(_f32),
        jnp.ones((N,), _f32),
        jnp.zeros((padn,), _f32),
    ])
    mbits = lax.bitcast_convert_type(maskf, _i32)
    e3 = jnp.concatenate([
        src.reshape(CHUNKS, KQ1, 128),
        dst.reshape(CHUNKS, KQ1, 128),
        mbits.reshape(CHUNKS, KQ1, 128),
    ], axis=1)

    eyeH = jnp.eye(H, dtype=_f32)
    Ms = (att_src1[0][:, :, None] * eyeH[:, None, :]).reshape(H * C, H)
    Md = (att_dst1[0][:, :, None] * eyeH[:, None, :]).reshape(H * C, H)
    A1 = jnp.concatenate([Ms, Md], axis=1)
    A2 = jnp.concatenate([
        jnp.tile(att_src2[0, 0][:, None], (1, H)),
        jnp.tile(att_dst2[0, 0][:, None], (1, H)),
    ], axis=1)
    E8 = jnp.repeat(eyeH, C, axis=1)
    b1r = b1.reshape(1, H * C)
    b2r = b2.reshape(1, NCLS)

    BN1 = 2000
    xl1, asd1 = pl.pallas_call(
        _dense1_body,
        grid=(N // BN1,),
        in_specs=[
            pl.BlockSpec((BN1, D), lambda i: (i, 0)),
            pl.BlockSpec((D, H * C), lambda i: (0, 0)),
            pl.BlockSpec((H * C, 2 * H), lambda i: (0, 0)),
        ],
        out_specs=[
            pl.BlockSpec((BN1, H * C), lambda i: (i, 0)),
            pl.BlockSpec((BN1, 2 * H), lambda i: (i, 0)),
        ],
        out_shape=[
            jax.ShapeDtypeStruct((N, H * C), _f32),
            jax.ShapeDtypeStruct((N, 2 * H), _f32),
        ],
    )(x, W1, A1)

    p64_1, p8_1 = _sc1(xl1, asd1, e3)

    BN2 = 2560
    xl2, asd2 = pl.pallas_call(
        _mid_body,
        grid=(NPAD // BN2,),
        in_specs=[
            pl.BlockSpec((2, BN2, 64), lambda i: (0, i, 0)),
            pl.BlockSpec((2, BN2, 16), lambda i: (0, i, 0)),
            pl.BlockSpec((1, H * C), lambda i: (0, 0)),
            pl.BlockSpec((H * C, NCLS), lambda i: (0, 0)),
            pl.BlockSpec((NCLS, 2 * H), lambda i: (0, 0)),
            pl.BlockSpec((H, H * C), lambda i: (0, 0)),
        ],
        out_specs=[
            pl.BlockSpec((BN2, NCLS), lambda i: (i, 0)),
            pl.BlockSpec((BN2, 2 * H), lambda i: (i, 0)),
        ],
        out_shape=[
            jax.ShapeDtypeStruct((NPAD, NCLS), _f32),
            jax.ShapeDtypeStruct((NPAD, 2 * H), _f32),
        ],
    )(p64_1, p8_1, b1r, W2, A2, E8)

    p64_2, p8_2 = _sc1(xl2[:N], asd2[:N], e3)

    out = pl.pallas_call(
        _fin_body,
        grid=(NPAD // BN2,),
        in_specs=[
            pl.BlockSpec((2, BN2, 64), lambda i: (0, i, 0)),
            pl.BlockSpec((2, BN2, 16), lambda i: (0, i, 0)),
            pl.BlockSpec((1, NCLS), lambda i: (0, 0)),
        ],
        out_specs=pl.BlockSpec((BN2, NCLS), lambda i: (i, 0)),
        out_shape=jax.ShapeDtypeStruct((NPAD, NCLS), _f32),
    )(p64_2, p8_2, b2r)

    return out[:N]
